# baseline (speedup 1.0000x reference)
import functools

import jax
import jax.numpy as jnp
from jax import lax
from jax.experimental import pallas as pl
from jax.experimental.pallas import tpu as pltpu
from jax.experimental.pallas import tpu_sc as plsc

_B, _K, _D, _N = 128, 32, 256, 8192
_R = 512
_BPS = _R // _K
_STEPS = (_B * _K) // _R

_NTILES = 16
_EPT = (_B * _K) // _NTILES
_BINS_PT = _N // _NTILES


def _argmin_body(x_ref, c_ref, amin_ref, w_ref, c2_ref):
    @pl.when(pl.program_id(0) == 0)
    def _init():
        c = c_ref[...]
        c2_ref[...] = lax.dot_general(
            jnp.full((8, _D), 0.5, jnp.float32), c * c,
            (((1,), (1,)), ((), ())),
            preferred_element_type=jnp.float32)

    x = x_ref[...]
    cross = lax.dot_general(
        x, c_ref[...], (((1,), (1,)), ((), ())),
        preferred_element_type=jnp.float32)
    score = cross - c2_ref[0:1, :]
    maxs = jnp.max(score, axis=1, keepdims=True)
    colidx = lax.broadcasted_iota(jnp.int32, (_R, _N), 1)
    amin = jnp.min(jnp.where(score >= maxs, colidx, _N), axis=1)
    amin_ref[...] = amin[None, None, :]

    a3 = amin.reshape(_BPS, _K)
    eq = a3[:, :, None] == a3[:, None, :]
    jj = lax.broadcasted_iota(jnp.int32, (_BPS, _K, _K), 1)
    ii = lax.broadcasted_iota(jnp.int32, (_BPS, _K, _K), 2)
    dupf = jnp.max(jnp.where(eq & (ii < jj), 1.0, 0.0), axis=2)
    w_ref[...] = (1.0 - dupf).reshape(1, 1, _R)


def _tc_argmin(x, centers):
    return pl.pallas_call(
        _argmin_body,
        grid=(_STEPS,),
        in_specs=[
            pl.BlockSpec((_R, _D), lambda i: (i, 0)),
            pl.BlockSpec((_N, _D), lambda i: (0, 0)),
        ],
        out_specs=[
            pl.BlockSpec((1, 1, _R), lambda i: (i, 0, 0)),
            pl.BlockSpec((1, 1, _R), lambda i: (i, 0, 0)),
        ],
        out_shape=[
            jax.ShapeDtypeStruct((_STEPS, 1, _R), jnp.int32),
            jax.ShapeDtypeStruct((_STEPS, 1, _R), jnp.float32),
        ],
        scratch_shapes=[pltpu.VMEM((8, _N), jnp.float32)],
    )(x, centers)


def _sc_hist(amin, w, zeros):
    mesh = plsc.VectorSubcoreMesh(core_axis_name="c", subcore_axis_name="s")

    @functools.partial(
        pl.kernel, mesh=mesh,
        out_type=jax.ShapeDtypeStruct((_N,), jnp.float32),
        scratch_types=[
            pltpu.VMEM((_EPT,), jnp.int32),
            pltpu.VMEM((_EPT,), jnp.float32),
            pltpu.VMEM_SHARED((_N,), jnp.float32),
        ],
    )
    def k(amin_hbm, w_hbm, zeros_hbm, out_hbm, idx_v, w_v, hist_sh):
        cid = lax.axis_index("c")
        sid = lax.axis_index("s")

        @pl.when(cid == 0)
        def _core0():
            base = sid * _EPT
            pltpu.sync_copy(amin_hbm.at[pl.ds(base, _EPT)], idx_v)
            pltpu.sync_copy(w_hbm.at[pl.ds(base, _EPT)], w_v)
            bins = sid * _BINS_PT
            pltpu.sync_copy(zeros_hbm.at[pl.ds(bins, _BINS_PT)],
                            hist_sh.at[pl.ds(bins, _BINS_PT)])
            plsc.subcore_barrier()
            pltpu.sync_copy(w_v, hist_sh.at[idx_v], add=True)
            plsc.subcore_barrier()
            pltpu.sync_copy(hist_sh.at[pl.ds(bins, _BINS_PT)],
                            out_hbm.at[pl.ds(bins, _BINS_PT)])

    return k(amin, w, zeros)


def _entropy_body(h_ref, out_ref):
    prob = h_ref[...] / (_B * _K)
    ent = -jnp.sum(prob * jnp.log(prob + 1e-10))
    out_ref[...] = ent[None, None]


def _tc_entropy(hist):
    return pl.pallas_call(
        _entropy_body,
        in_specs=[pl.BlockSpec((64, 128), lambda: (0, 0))],
        out_specs=pl.BlockSpec((1, 1), lambda: (0, 0)),
        out_shape=jax.ShapeDtypeStruct((1, 1), jnp.float32),
    )(hist)


def kernel(selected_frames, cluster_centers):
    x = selected_frames.reshape(_B * _K, _D)
    amin, w = _tc_argmin(x, cluster_centers)
    zeros = jnp.zeros((_N,), jnp.float32)
    hist = _sc_hist(amin.reshape(_B * _K), w.reshape(_B * _K), zeros)
    out = _tc_entropy(hist.reshape(64, 128))
    return out[0, 0]


# speedup vs baseline: 1.4391x; 1.4391x over previous
import jax
import jax.numpy as jnp
from jax import lax
from jax.experimental import pallas as pl
from jax.experimental.pallas import tpu as pltpu

_B, _K, _D, _N = 128, 32, 256, 8192
_R = 512
_BPS = _R // _K
_STEPS = (_B * _K) // _R
_NC = 4
_CN = _N // _NC


def _phase1(x_ref, c_ref, c2_ref, s_ref, mx_ref):
    x = x_ref[...]
    m = None
    for t in range(_NC):
        cols = slice(t * _CN, (t + 1) * _CN)
        sc = lax.dot_general(
            x, c_ref[cols, :], (((1,), (1,)), ((), ())),
            preferred_element_type=jnp.float32)
        sc = sc - c2_ref[0:1, cols]
        s_ref[:, cols] = sc
        mt = jnp.max(sc, axis=1, keepdims=True)
        m = mt if m is None else jnp.maximum(m, mt)
    mx_ref[...] = jnp.broadcast_to(m, (_R, 128))


def _phase2(s_ref, mx_ref, cov_ref):
    g = s_ref[...] - mx_ref[:, 0:1]
    for b in range(_BPS):
        gb = jnp.max(g[b * _K:(b + 1) * _K, :], axis=0, keepdims=True)
        cov_ref[b:b + 1, :] += jnp.where(gb >= 0.0, 1.0, 0.0)


def _cluster_body(x_ref, c_ref, out_ref, cov_ref, c2_ref,
                  s0_ref, s1_ref, mx0_ref, mx1_ref):
    step = pl.program_id(0)

    @pl.when(step == 0)
    def _first():
        cov_ref[...] = jnp.zeros_like(cov_ref)
        c = c_ref[...]
        c2_ref[...] = lax.dot_general(
            jnp.full((8, _D), 0.5, jnp.float32), c * c,
            (((1,), (1,)), ((), ())),
            preferred_element_type=jnp.float32)
        _phase1(x_ref, c_ref, c2_ref, s0_ref, mx0_ref)

    @pl.when((step > 0) & (step < _STEPS) & (step % 2 == 1))
    def _mid_odd():
        _phase1(x_ref, c_ref, c2_ref, s1_ref, mx1_ref)
        _phase2(s0_ref, mx0_ref, cov_ref)

    @pl.when((step > 0) & (step < _STEPS) & (step % 2 == 0))
    def _mid_even():
        _phase1(x_ref, c_ref, c2_ref, s0_ref, mx0_ref)
        _phase2(s1_ref, mx1_ref, cov_ref)

    @pl.when(step == _STEPS)
    def _last():
        _phase2(s1_ref, mx1_ref, cov_ref)
        coverage = jnp.sum(cov_ref[...], axis=0, keepdims=True)
        prob = coverage / (_B * _K)
        ent = -jnp.sum(prob * jnp.log(prob + 1e-10))
        out_ref[...] = ent[None, None]


def kernel(selected_frames, cluster_centers):
    x = selected_frames.reshape(_B * _K, _D)
    out = pl.pallas_call(
        _cluster_body,
        grid=(_STEPS + 1,),
        in_specs=[
            pl.BlockSpec((_R, _D), lambda i: (jnp.minimum(i, _STEPS - 1), 0)),
            pl.BlockSpec((_N, _D), lambda i: (0, 0)),
        ],
        out_specs=pl.BlockSpec((1, 1), lambda i: (0, 0)),
        out_shape=jax.ShapeDtypeStruct((1, 1), jnp.float32),
        scratch_shapes=[
            pltpu.VMEM((_BPS, _N), jnp.float32),
            pltpu.VMEM((8, _N), jnp.float32),
            pltpu.VMEM((_R, _N), jnp.float32),
            pltpu.VMEM((_R, _N), jnp.float32),
            pltpu.VMEM((_R, 128), jnp.float32),
            pltpu.VMEM((_R, 128), jnp.float32),
        ],
    )(x, cluster_centers)
    return out[0, 0]


# speedup vs baseline: 1.5670x; 1.0889x over previous
import jax
import jax.numpy as jnp
from jax import lax
from jax.experimental import pallas as pl
from jax.experimental.pallas import tpu as pltpu

_B, _K, _D, _N = 128, 32, 256, 8192
_R = 1024
_BPS = _R // _K
_STEPS = (_B * _K) // _R
_NC = 8
_CN = _N // _NC


def _cluster_body(x_ref, c_ref, out_ref, cov_ref, c2_ref, s_ref):
    step = pl.program_id(0)

    @pl.when(step == 0)
    def _init():
        cov_ref[...] = jnp.zeros_like(cov_ref)
        c = c_ref[...]
        c2_ref[...] = lax.dot_general(
            jnp.full((8, _D), 0.5, jnp.float32), c * c,
            (((1,), (1,)), ((), ())),
            preferred_element_type=jnp.float32)

    x = x_ref[...]
    m = None
    for t in range(_NC):
        cols = slice(t * _CN, (t + 1) * _CN)
        sc = lax.dot_general(
            x, c_ref[cols, :], (((1,), (1,)), ((), ())),
            preferred_element_type=jnp.float32)
        sc = sc - c2_ref[0:1, cols]
        s_ref[:, cols] = sc
        mt = jnp.max(sc, axis=1, keepdims=True)
        m = mt if m is None else jnp.maximum(m, mt)

    g = s_ref[...] - m
    for b in range(_BPS):
        gb = jnp.max(g[b * _K:(b + 1) * _K, :], axis=0, keepdims=True)
        cov_ref[b:b + 1, :] += jnp.where(gb >= 0.0, 1.0, 0.0)

    @pl.when(step == _STEPS - 1)
    def _fini():
        coverage = jnp.sum(cov_ref[...], axis=0, keepdims=True)
        prob = coverage / (_B * _K)
        ent = -jnp.sum(prob * jnp.log(prob + 1e-10))
        out_ref[...] = ent[None, None]


def kernel(selected_frames, cluster_centers):
    x = selected_frames.reshape(_B * _K, _D)
    out = pl.pallas_call(
        _cluster_body,
        grid=(_STEPS,),
        in_specs=[
            pl.BlockSpec((_R, _D), lambda i: (i, 0)),
            pl.BlockSpec((_N, _D), lambda i: (0, 0)),
        ],
        out_specs=pl.BlockSpec((1, 1), lambda i: (0, 0)),
        out_shape=jax.ShapeDtypeStruct((1, 1), jnp.float32),
        scratch_shapes=[
            pltpu.VMEM((_BPS, _N), jnp.float32),
            pltpu.VMEM((8, _N), jnp.float32),
            pltpu.VMEM((_R, _N), jnp.float32),
        ],
    )(x, cluster_centers)
    return out[0, 0]


# speedup vs baseline: 1.6696x; 1.0655x over previous
import jax
import jax.numpy as jnp
from jax import lax
from jax.experimental import pallas as pl
from jax.experimental.pallas import tpu as pltpu

_B, _K, _D, _N = 128, 32, 256, 8192
_R = 1024
_BPS = _R // _K
_STEPS = (_B * _K) // _R
_NC = 32
_CN = _N // _NC


def _cluster_body(x_ref, c_ref, out_ref, cov_ref, c2_ref, s_ref):
    step = pl.program_id(0)

    @pl.when(step == 0)
    def _init():
        cov_ref[...] = jnp.zeros_like(cov_ref)
        c = c_ref[...]
        c2_ref[...] = lax.dot_general(
            jnp.full((8, _D), 0.5, jnp.float32), c * c,
            (((1,), (1,)), ((), ())),
            preferred_element_type=jnp.float32)

    x = x_ref[...]
    m = None
    for t in range(_NC):
        cols = slice(t * _CN, (t + 1) * _CN)
        sc = lax.dot_general(
            x, c_ref[cols, :], (((1,), (1,)), ((), ())),
            preferred_element_type=jnp.float32)
        sc = sc - c2_ref[0:1, cols]
        s_ref[:, cols] = sc
        mt = jnp.max(sc, axis=1, keepdims=True)
        m = mt if m is None else jnp.maximum(m, mt)

    g = s_ref[...] - m
    for b in range(_BPS):
        gb = jnp.max(g[b * _K:(b + 1) * _K, :], axis=0, keepdims=True)
        cov_ref[b:b + 1, :] += jnp.where(gb >= 0.0, 1.0, 0.0)

    @pl.when(step == _STEPS - 1)
    def _fini():
        coverage = jnp.sum(cov_ref[...], axis=0, keepdims=True)
        prob = coverage / (_B * _K)
        ent = -jnp.sum(prob * jnp.log(prob + 1e-10))
        out_ref[...] = ent[None, None]


def kernel(selected_frames, cluster_centers):
    x = selected_frames.reshape(_B * _K, _D)
    out = pl.pallas_call(
        _cluster_body,
        grid=(_STEPS,),
        in_specs=[
            pl.BlockSpec((_R, _D), lambda i: (i, 0)),
            pl.BlockSpec((_N, _D), lambda i: (0, 0)),
        ],
        out_specs=pl.BlockSpec((1, 1), lambda i: (0, 0)),
        out_shape=jax.ShapeDtypeStruct((1, 1), jnp.float32),
        scratch_shapes=[
            pltpu.VMEM((_BPS, _N), jnp.float32),
            pltpu.VMEM((8, _N), jnp.float32),
            pltpu.VMEM((_R, _N), jnp.float32),
        ],
    )(x, cluster_centers)
    return out[0, 0]
